# baseline (speedup 1.0000x reference)
import functools

import jax
import jax.numpy as jnp
from jax import lax
from jax.experimental import pallas as pl
from jax.experimental.pallas import tpu as pltpu
from jax.experimental.pallas import tpu_sc as plsc

N_NODES = 10000
NPAD = 10240
N_EDGES = 320000
D_IN = 128
D_H1 = 128
D_H2 = 64

NC = 2
NS = 16
NW = NC * NS
CHUNK = 128
NCHUNK = 80
EPW = NCHUNK * CHUNK
E_PAD = NW * EPW
ZONE = NPAD // NS

_mesh = lambda: plsc.VectorSubcoreMesh(core_axis_name="c", subcore_axis_name="s")


def _zero_rows(rows, nrow, d):
    zero_pat = jnp.zeros((16,), jnp.float32)

    def zstep(i, carry):
        for j in range(d // 16):
            rows[i, pl.ds(j * 16, 16)] = zero_pat
        return carry

    lax.fori_loop(0, nrow, zstep, 0)


def _zero_zone(rows, acc_sh, s):
    for i in range(ZONE // CHUNK):
        pltpu.sync_copy(rows, acc_sh.at[pl.ds(s * ZONE + i * CHUNK, CHUNK)])


def _zone_out(acc_sh, rows, out_hbm, c, s):
    for i in range(ZONE // CHUNK):
        off = s * ZONE + i * CHUNK
        pltpu.sync_copy(acc_sh.at[pl.ds(off, CHUNK)], rows)
        pltpu.sync_copy(rows, out_hbm.at[c, pl.ds(off, CHUNK)])


HW = 8


def _hist_body(dst_hbm, ones_hbm, zeros_hbm, out_hbm, dstv, obuf, zbuf, acc_sh, sem):
    c = lax.axis_index("c")
    s = lax.axis_index("s")
    w = c * NS + s
    idx_cp = pltpu.make_async_copy(dst_hbm.at[w], dstv, sem)
    idx_cp.start()
    pltpu.sync_copy(ones_hbm, obuf)
    pltpu.sync_copy(zeros_hbm, zbuf)
    _zero_zone(zbuf, acc_sh, s)
    idx_cp.wait()
    plsc.subcore_barrier()

    def step(k, carry):
        pltpu.async_copy(obuf, acc_sh.at[dstv.at[k]], sem, add=True)
        return carry

    lax.fori_loop(0, NCHUNK, step, 0)

    def drain(k, carry):
        pltpu.make_async_copy(obuf, acc_sh.at[dstv.at[k]], sem).wait()
        return carry

    lax.fori_loop(0, NCHUNK, drain, 0)
    plsc.subcore_barrier()
    _zone_out(acc_sh, zbuf, out_hbm, c, s)


def _make_hist():
    return pl.kernel(
        _hist_body,
        out_type=jax.ShapeDtypeStruct((NC, NPAD, HW), jnp.float32),
        mesh=_mesh(),
        compiler_params=pltpu.CompilerParams(use_tc_tiling_on_sc=False),
        scratch_types=[
            pltpu.VMEM((NCHUNK, CHUNK), jnp.int32),
            pltpu.VMEM((CHUNK, HW), jnp.float32),
            pltpu.VMEM((CHUNK, HW), jnp.float32),
            pltpu.VMEM_SHARED((NPAD, HW), jnp.float32),
            pltpu.SemaphoreType.DMA,
        ],
    )


DAGG = 64


def _agg_body(npass, *refs):
    gs = refs[:npass]
    src_hbm, dst_hbm, out_hbm = refs[npass:npass + 3]
    srcv, dstv, rows0, rows1, acc_sh, isem, gsem0, gsem1 = refs[npass + 3:]
    c = lax.axis_index("c")
    s = lax.axis_index("s")
    w = c * NS + s
    src_cp = pltpu.make_async_copy(src_hbm.at[w], srcv, isem)
    dst_cp = pltpu.make_async_copy(dst_hbm.at[w], dstv, isem)
    src_cp.start()
    dst_cp.start()
    src_cp.wait()
    dst_cp.wait()

    bufs = (rows0, rows1)
    sems = (gsem0, gsem1)
    for pi, g_hbm in enumerate(gs):
        _zero_rows(rows0, CHUNK, DAGG)
        _zero_zone(rows0, acc_sh, s)
        plsc.subcore_barrier()

        pltpu.async_copy(g_hbm.at[srcv.at[0]], rows0, gsem0)

        def pair(p, carry):
            for b in range(2):
                k = 2 * p + b
                @pl.when(k < NCHUNK - 1)
                def _():
                    pltpu.async_copy(g_hbm.at[srcv.at[k + 1]], bufs[1 - b], sems[1 - b])
                pltpu.make_async_copy(g_hbm.at[srcv.at[k]], bufs[b], sems[b]).wait()
                pltpu.sync_copy(bufs[b], acc_sh.at[dstv.at[k]], add=True)
            return carry

        lax.fori_loop(0, NCHUNK // 2, pair, 0)
        plsc.subcore_barrier()
        for i in range(ZONE // CHUNK):
            off = s * ZONE + i * CHUNK
            pltpu.sync_copy(acc_sh.at[pl.ds(off, CHUNK)], rows1)
            pltpu.sync_copy(rows1, out_hbm.at[c, pi, pl.ds(off, CHUNK)])
        plsc.subcore_barrier()


def _make_agg(npass):
    return pl.kernel(
        functools.partial(_agg_body, npass),
        out_type=jax.ShapeDtypeStruct((NC, npass, NPAD, DAGG), jnp.float32),
        mesh=_mesh(),
        compiler_params=pltpu.CompilerParams(use_tc_tiling_on_sc=False),
        scratch_types=[
            pltpu.VMEM((NCHUNK, CHUNK), jnp.int32),
            pltpu.VMEM((NCHUNK, CHUNK), jnp.int32),
            pltpu.VMEM((CHUNK, DAGG), jnp.float32),
            pltpu.VMEM((CHUNK, DAGG), jnp.float32),
            pltpu.VMEM_SHARED((NPAD, DAGG), jnp.float32),
            pltpu.SemaphoreType.DMA,
            pltpu.SemaphoreType.DMA,
            pltpu.SemaphoreType.DMA,
        ],
    )


BLK = 512
GRID = NPAD // BLK


def _tc_a_body(x_ref, wa_ref, wb_ref, h0_ref, h1_ref, ga_ref, gb_ref, dis_ref):
    deg = 1.0 + h0_ref[:, 0:1] + h1_ref[:, 0:1]
    dis = lax.rsqrt(deg)
    ga_ref[...] = dis * jnp.dot(x_ref[...], wa_ref[...], preferred_element_type=jnp.float32)
    gb_ref[...] = dis * jnp.dot(x_ref[...], wb_ref[...], preferred_element_type=jnp.float32)
    dis_ref[...] = dis


def _tc_b_body(a00_ref, a01_ref, a10_ref, a11_ref, ga_ref, gb_ref, dis_ref,
               b_ref, w_ref, out_ref):
    dis = dis_ref[...]
    o1a = dis * (a00_ref[...] + a10_ref[...] + ga_ref[...])
    o1b = dis * (a01_ref[...] + a11_ref[...] + gb_ref[...])
    o1 = jnp.maximum(jnp.concatenate([o1a, o1b], axis=1) + b_ref[...], 0.0)
    out_ref[...] = dis * jnp.dot(o1, w_ref[...], preferred_element_type=jnp.float32)


def _tc_c_body(a0_ref, a1_ref, g_ref, dis_ref, b_ref, out_ref):
    dis = dis_ref[...]
    out_ref[...] = jnp.maximum(
        dis * (a0_ref[...] + a1_ref[...] + g_ref[...]) + b_ref[...], 0.0)


def _row_spec(d):
    return pl.BlockSpec((BLK, d), lambda i: (i, 0))


def _full_spec(r, c):
    return pl.BlockSpec((r, c), lambda i: (0, 0))


_tc_a = pl.pallas_call(
    _tc_a_body,
    grid=(GRID,),
    in_specs=[_row_spec(D_IN), _full_spec(D_IN, DAGG), _full_spec(D_IN, DAGG),
              _row_spec(HW), _row_spec(HW)],
    out_specs=[_row_spec(DAGG), _row_spec(DAGG), _row_spec(1)],
    out_shape=[jax.ShapeDtypeStruct((NPAD, DAGG), jnp.float32),
               jax.ShapeDtypeStruct((NPAD, DAGG), jnp.float32),
               jax.ShapeDtypeStruct((NPAD, 1), jnp.float32)],
)

_tc_b = pl.pallas_call(
    _tc_b_body,
    grid=(GRID,),
    in_specs=[_row_spec(DAGG)] * 6 + [_row_spec(1),
              _full_spec(1, D_H1), _full_spec(D_H1, D_H2)],
    out_specs=_row_spec(D_H2),
    out_shape=jax.ShapeDtypeStruct((NPAD, D_H2), jnp.float32),
)

_tc_c = pl.pallas_call(
    _tc_c_body,
    grid=(GRID,),
    in_specs=[_row_spec(D_H2), _row_spec(D_H2), _row_spec(D_H2), _row_spec(1),
              _full_spec(1, D_H2)],
    out_specs=_row_spec(D_H2),
    out_shape=jax.ShapeDtypeStruct((NPAD, D_H2), jnp.float32),
)

_hist = _make_hist()
_agg1 = _make_agg(2)
_agg2 = _make_agg(1)


def kernel(x, edge_index, W1, b1, W2, b2):
    ei = edge_index.astype(jnp.int32)
    pad = jnp.full((E_PAD - N_EDGES,), NPAD - 1, jnp.int32)
    src = jnp.concatenate([ei[0], pad]).reshape(NW, NCHUNK, CHUNK)
    dst = jnp.concatenate([ei[1], pad]).reshape(NW, NCHUNK, CHUNK)
    x_pad = jnp.pad(x, ((0, NPAD - N_NODES), (0, 0)))

    ones8 = jnp.tile(jnp.eye(1, HW, dtype=jnp.float32), (CHUNK, 1))
    zeros8 = jnp.zeros((CHUNK, HW), jnp.float32)
    hist = _hist(dst, ones8, zeros8)
    g1a, g1b, dis = _tc_a(x_pad, W1[:, :DAGG], W1[:, DAGG:], hist[0], hist[1])
    acc1 = _agg1(g1a, g1b, src, dst)
    g2 = _tc_b(acc1[0, 0], acc1[0, 1], acc1[1, 0], acc1[1, 1], g1a, g1b, dis,
               b1.reshape(1, -1), W2)
    acc2 = _agg2(g2, src, dst)
    out = _tc_c(acc2[0, 0], acc2[1, 0], g2, dis, b2.reshape(1, -1))
    return out[:N_NODES]


# speedup vs baseline: 2.5252x; 2.5252x over previous
import functools

import jax
import jax.numpy as jnp
from jax import lax
from jax.experimental import pallas as pl
from jax.experimental.pallas import tpu as pltpu
from jax.experimental.pallas import tpu_sc as plsc

N_NODES = 10000
NPAD = 10240
N_EDGES = 320000
D_IN = 128
D_H1 = 128
D_H2 = 64

NC = 2
NS = 16
NW = NC * NS
CHUNK = 128
NCHUNK = 80
EPW = NCHUNK * CHUNK
E_PAD = NW * EPW
ZONE = NPAD // NS

_mesh = lambda: plsc.VectorSubcoreMesh(core_axis_name="c", subcore_axis_name="s")


def _zero_rows(rows, nrow, d):
    zero_pat = jnp.zeros((16,), jnp.float32)

    def zstep(i, carry):
        for j in range(d // 16):
            rows[i, pl.ds(j * 16, 16)] = zero_pat
        return carry

    lax.fori_loop(0, nrow, zstep, 0)


def _zero_zone(rows, acc_sh, s):
    for i in range(ZONE // CHUNK):
        pltpu.sync_copy(rows, acc_sh.at[pl.ds(s * ZONE + i * CHUNK, CHUNK)])


def _zone_out(acc_sh, rows, out_hbm, c, s):
    for i in range(ZONE // CHUNK):
        off = s * ZONE + i * CHUNK
        pltpu.sync_copy(acc_sh.at[pl.ds(off, CHUNK)], rows)
        pltpu.sync_copy(rows, out_hbm.at[c, pl.ds(off, CHUNK)])


HW = 8


def _hist_body(dst_hbm, ones_hbm, zeros_hbm, out_hbm, dstv, obuf, zbuf, acc_sh, sem):
    c = lax.axis_index("c")
    s = lax.axis_index("s")
    w = c * NS + s
    idx_cp = pltpu.make_async_copy(dst_hbm.at[w], dstv, sem)
    idx_cp.start()
    pltpu.sync_copy(ones_hbm, obuf)
    pltpu.sync_copy(zeros_hbm, zbuf)
    _zero_zone(zbuf, acc_sh, s)
    idx_cp.wait()
    plsc.subcore_barrier()

    def step(k, carry):
        pltpu.async_copy(obuf, acc_sh.at[dstv.at[k]], sem, add=True)
        return carry

    lax.fori_loop(0, NCHUNK, step, 0)

    def drain(k, carry):
        pltpu.make_async_copy(obuf, acc_sh.at[dstv.at[k]], sem).wait()
        return carry

    lax.fori_loop(0, NCHUNK, drain, 0)
    plsc.subcore_barrier()
    _zone_out(acc_sh, zbuf, out_hbm, c, s)


def _make_hist():
    return pl.kernel(
        _hist_body,
        out_type=jax.ShapeDtypeStruct((NC, NPAD, HW), jnp.float32),
        mesh=_mesh(),
        compiler_params=pltpu.CompilerParams(use_tc_tiling_on_sc=False),
        scratch_types=[
            pltpu.VMEM((NCHUNK, CHUNK), jnp.int32),
            pltpu.VMEM((CHUNK, HW), jnp.float32),
            pltpu.VMEM((CHUNK, HW), jnp.float32),
            pltpu.VMEM_SHARED((NPAD, HW), jnp.float32),
            pltpu.SemaphoreType.DMA,
        ],
    )


DAGG = 64


def _agg_body(npass, *refs):
    gs = refs[:npass]
    src_hbm, dst_hbm, out_hbm = refs[npass:npass + 3]
    srcv, dstv, rows0, rows1, acc_sh, isem, gsem0, gsem1 = refs[npass + 3:]
    c = lax.axis_index("c")
    s = lax.axis_index("s")
    w = c * NS + s
    src_cp = pltpu.make_async_copy(src_hbm.at[w], srcv, isem)
    dst_cp = pltpu.make_async_copy(dst_hbm.at[w], dstv, isem)
    src_cp.start()
    dst_cp.start()
    src_cp.wait()
    dst_cp.wait()

    bufs = (rows0, rows1)
    sems = (gsem0, gsem1)
    for pi, g_hbm in enumerate(gs):
        _zero_rows(rows0, CHUNK, DAGG)
        _zero_zone(rows0, acc_sh, s)
        plsc.subcore_barrier()

        pltpu.async_copy(g_hbm.at[srcv.at[0]], rows0, gsem0)

        def pair(p, carry):
            for b in range(2):
                k = 2 * p + b
                @pl.when(k < NCHUNK - 1)
                def _():
                    pltpu.async_copy(g_hbm.at[srcv.at[k + 1]], bufs[1 - b], sems[1 - b])
                pltpu.make_async_copy(g_hbm.at[srcv.at[k]], bufs[b], sems[b]).wait()
                pltpu.sync_copy(bufs[b], acc_sh.at[dstv.at[k]], add=True)
            return carry

        lax.fori_loop(0, NCHUNK // 2, pair, 0)
        plsc.subcore_barrier()
        for i in range(ZONE // CHUNK):
            off = s * ZONE + i * CHUNK
            pltpu.sync_copy(acc_sh.at[pl.ds(off, CHUNK)], rows1)
            pltpu.sync_copy(rows1, out_hbm.at[c, pi, pl.ds(off, CHUNK)])
        plsc.subcore_barrier()


def _make_agg(npass):
    return pl.kernel(
        functools.partial(_agg_body, npass),
        out_type=jax.ShapeDtypeStruct((NC, npass, NPAD, DAGG), jnp.float32),
        mesh=_mesh(),
        compiler_params=pltpu.CompilerParams(use_tc_tiling_on_sc=False),
        scratch_types=[
            pltpu.VMEM((NCHUNK, CHUNK), jnp.int32),
            pltpu.VMEM((NCHUNK, CHUNK), jnp.int32),
            pltpu.VMEM((CHUNK, DAGG), jnp.float32),
            pltpu.VMEM((CHUNK, DAGG), jnp.float32),
            pltpu.VMEM_SHARED((NPAD, DAGG), jnp.float32),
            pltpu.SemaphoreType.DMA,
            pltpu.SemaphoreType.DMA,
            pltpu.SemaphoreType.DMA,
        ],
    )


BLK = 512
GRID = NPAD // BLK


def _tc_a_body(x_ref, wa_ref, wb_ref, h0_ref, h1_ref, ga_ref, gb_ref, dis_ref):
    deg = 1.0 + h0_ref[:, 0:1] + h1_ref[:, 0:1]
    dis = lax.rsqrt(deg)
    ga_ref[...] = dis * jnp.dot(x_ref[...], wa_ref[...], preferred_element_type=jnp.float32)
    gb_ref[...] = dis * jnp.dot(x_ref[...], wb_ref[...], preferred_element_type=jnp.float32)
    dis_ref[...] = dis


def _tc_b_body(a00_ref, a01_ref, a10_ref, a11_ref, ga_ref, gb_ref, dis_ref,
               b_ref, w_ref, out_ref):
    dis = dis_ref[...]
    o1a = dis * (a00_ref[...] + a10_ref[...] + ga_ref[...])
    o1b = dis * (a01_ref[...] + a11_ref[...] + gb_ref[...])
    o1 = jnp.maximum(jnp.concatenate([o1a, o1b], axis=1) + b_ref[...], 0.0)
    out_ref[...] = dis * jnp.dot(o1, w_ref[...], preferred_element_type=jnp.float32)


def _tc_c_body(a0_ref, a1_ref, g_ref, dis_ref, b_ref, out_ref):
    dis = dis_ref[...]
    out_ref[...] = jnp.maximum(
        dis * (a0_ref[...] + a1_ref[...] + g_ref[...]) + b_ref[...], 0.0)


def _row_spec(d):
    return pl.BlockSpec((BLK, d), lambda i: (i, 0))


def _full_spec(r, c):
    return pl.BlockSpec((r, c), lambda i: (0, 0))


_tc_a = pl.pallas_call(
    _tc_a_body,
    grid=(GRID,),
    in_specs=[_row_spec(D_IN), _full_spec(D_IN, DAGG), _full_spec(D_IN, DAGG),
              _row_spec(HW), _row_spec(HW)],
    out_specs=[_row_spec(DAGG), _row_spec(DAGG), _row_spec(1)],
    out_shape=[jax.ShapeDtypeStruct((NPAD, DAGG), jnp.float32),
               jax.ShapeDtypeStruct((NPAD, DAGG), jnp.float32),
               jax.ShapeDtypeStruct((NPAD, 1), jnp.float32)],
)

_tc_b = pl.pallas_call(
    _tc_b_body,
    grid=(GRID,),
    in_specs=[_row_spec(DAGG)] * 6 + [_row_spec(1),
              _full_spec(1, D_H1), _full_spec(D_H1, D_H2)],
    out_specs=_row_spec(D_H2),
    out_shape=jax.ShapeDtypeStruct((NPAD, D_H2), jnp.float32),
)

_tc_c = pl.pallas_call(
    _tc_c_body,
    grid=(GRID,),
    in_specs=[_row_spec(D_H2), _row_spec(D_H2), _row_spec(D_H2), _row_spec(1),
              _full_spec(1, D_H2)],
    out_specs=_row_spec(D_H2),
    out_shape=jax.ShapeDtypeStruct((NPAD, D_H2), jnp.float32),
)

_hist = _make_hist()
_agg1 = _make_agg(2)
_agg2 = _make_agg(1)


def kernel(x, edge_index, W1, b1, W2, b2):
    ei = edge_index.astype(jnp.int32)
    pad = N_NODES + jnp.arange(E_PAD - N_EDGES, dtype=jnp.int32) % (NPAD - N_NODES)
    src = jnp.concatenate([ei[0], pad]).reshape(NW, NCHUNK, CHUNK)
    dst = jnp.concatenate([ei[1], pad]).reshape(NW, NCHUNK, CHUNK)
    x_pad = jnp.pad(x, ((0, NPAD - N_NODES), (0, 0)))

    ones8 = jnp.tile(jnp.eye(1, HW, dtype=jnp.float32), (CHUNK, 1))
    zeros8 = jnp.zeros((CHUNK, HW), jnp.float32)
    hist = _hist(dst, ones8, zeros8)
    g1a, g1b, dis = _tc_a(x_pad, W1[:, :DAGG], W1[:, DAGG:], hist[0], hist[1])
    acc1 = _agg1(g1a, g1b, src, dst)
    g2 = _tc_b(acc1[0, 0], acc1[0, 1], acc1[1, 0], acc1[1, 1], g1a, g1b, dis,
               b1.reshape(1, -1), W2)
    acc2 = _agg2(g2, src, dst)
    out = _tc_c(acc2[0, 0], acc2[1, 0], g2, dis, b2.reshape(1, -1))
    return out[:N_NODES]
